# baseline (speedup 1.0000x reference)
import math

import jax
import jax.numpy as jnp
from jax.experimental import pallas as pl
from jax.experimental.pallas import tpu as pltpu

B, T, D = 1, 2048, 768
H, HD = 12, 64
G, R, WIN = 16, 4, 64
NB = T // G
SCALE = math.sqrt(HD)
HALF = WIN // 2
TQ = 1024
TK = 256
PRE = 128
BAND = TQ + PRE
NEG_TOPK = -1.0e30
NEG_TAKEN = -3.0e38
NEG_MASK = -3.0e38
BIG = 2.0 ** 100

_INTERPRET = False


def _safe_sm(s):
    m = jnp.max(s, axis=-1, keepdims=True)
    m = jnp.where(m > -jnp.inf, m, 0.0)
    e = jnp.exp(s - m)
    z = jnp.sum(e, axis=-1, keepdims=True)
    return e / jnp.where(z == 0.0, 1.0, z)


def _qkv_gates_kernel(x_ref, w_ref, wg_ref, bg_ref,
                      q_ref, k_ref, v_ref, g_ref):
    qkv = jax.lax.dot_general(x_ref[...], w_ref[...], (((1,), (0,)), ((), ())),
                              preferred_element_type=jnp.float32)
    qm = jnp.zeros((TQ, HD), jnp.float32)
    for h in range(H):
        qh = qkv[:, h * HD:(h + 1) * HD]
        q_ref[h] = qh
        qm = qm + qh
        k_ref[h] = qkv[:, D + h * HD:D + (h + 1) * HD]
        v_ref[h] = qkv[:, 2 * D + h * HD:2 * D + (h + 1) * HD]
    qm = qm * (1.0 / H)
    glog = jax.lax.dot_general(qm, wg_ref[...], (((1,), (1,)), ((), ())),
                               preferred_element_type=jnp.float32)
    glog = glog + bg_ref[...]
    m = jnp.max(glog, axis=-1, keepdims=True)
    e = jnp.exp(glog - m)
    g_ref[...] = e / jnp.sum(e, axis=-1, keepdims=True)


def _attn_kernel(q_ref, k_ref, v_ref, g_ref, wck_ref, wcv_ref, e_ref,
                 out_ref, kc_s, vc_s):
    qc = pl.program_id(1)

    @pl.when(qc == 0)
    def _compress():
        k3 = k_ref[0].reshape(NB, G, HD)
        v3 = v_ref[0].reshape(NB, G, HD)
        kc = jnp.zeros((NB, HD), jnp.float32)
        vc = jnp.zeros((NB, HD), jnp.float32)
        for g in range(G):
            wck_g = wck_ref[:, g * HD:(g + 1) * HD]
            wcv_g = wcv_ref[:, g * HD:(g + 1) * HD]
            kc = kc + jax.lax.dot_general(k3[:, g, :], wck_g,
                                          (((1,), (1,)), ((), ())),
                                          preferred_element_type=jnp.float32)
            vc = vc + jax.lax.dot_general(v3[:, g, :], wcv_g,
                                          (((1,), (1,)), ((), ())),
                                          preferred_element_type=jnp.float32)
        kc_s[...] = kc
        vc_s[...] = vc

    q = q_ref[0]
    qs = q * (1.0 / SCALE)
    sc = jax.lax.dot_general(qs, kc_s[...], (((1,), (1,)), ((), ())),
                             preferred_element_type=jnp.float32)
    bi = jax.lax.broadcasted_iota(jnp.int32, (TQ, NB), 1)
    ri = jax.lax.broadcasted_iota(jnp.int32, (TQ, NB), 0) + qc * TQ
    valid = ri >= (bi + 1) * G
    sc_m = jnp.where(valid, sc, -jnp.inf)
    mc = jnp.max(sc_m, axis=-1, keepdims=True)
    mc = jnp.where(mc > -jnp.inf, mc, 0.0)
    e_c = jnp.exp(sc_m - mc)
    z_c = jnp.sum(e_c, axis=-1, keepdims=True)
    u_c = jax.lax.dot_general(e_c, vc_s[...], (((1,), (0,)), ((), ())),
                              preferred_element_type=jnp.float32)
    s = jnp.where(valid, sc,
                  NEG_TOPK * (1.0 + bi.astype(jnp.float32) * (1.0 / 1024.0)))
    bmask = jnp.zeros((TQ, NB), jnp.float32)
    for _ in range(R):
        m = jnp.max(s, axis=-1, keepdims=True)
        hit = s >= m
        bmask = jnp.where(hit, 1.0, bmask)
        s = jnp.where(hit, NEG_TAKEN, s)
    start = jnp.maximum(qc * TQ - PRE, 0)
    k_b = k_ref[0, pl.ds(start, BAND), :]
    v_b = v_ref[0, pl.ds(start, BAND), :]
    s_b = jax.lax.dot_general(qs, k_b, (((1,), (1,)), ((), ())),
                              preferred_element_type=jnp.float32)
    ci = jax.lax.broadcasted_iota(jnp.int32, (TQ, BAND), 1) + start
    rw = jax.lax.broadcasted_iota(jnp.int32, (TQ, BAND), 0) + qc * TQ
    s_b = jnp.where((ci <= rw) & (ci >= rw - HALF), s_b, NEG_MASK)
    m_l = jnp.max(s_b, axis=-1, keepdims=True)
    e_b = jnp.exp(s_b - m_l)
    z_l = jnp.sum(e_b, axis=-1, keepdims=True)
    out_l = jax.lax.dot_general(e_b, v_b, (((1,), (0,)), ((), ())),
                                preferred_element_type=jnp.float32)
    g = g_ref[...]
    alpha = g[:, 0:1] / jnp.where(z_c == 0.0, 1.0, z_c)
    g_beta = g[:, 1:2]
    gamma_out_l = (g[:, 2:3] / z_l) * out_l
    rest = alpha * u_c + gamma_out_l

    def _selected(width, fill_causal):
        kw = k_ref[0, 0:width, :]
        vw = v_ref[0, 0:width, :]
        s_w = jax.lax.dot_general(qs, kw, (((1,), (1,)), ((), ())),
                                  preferred_element_type=jnp.float32)
        selm = jax.lax.dot_general(bmask, e_ref[:, 0:width],
                                   (((1,), (0,)), ((), ())),
                                   preferred_element_type=jnp.float32)
        cond = selm > 0.5
        if fill_causal:
            ti = jax.lax.broadcasted_iota(jnp.int32, (TQ, width), 1)
            tr = jax.lax.broadcasted_iota(jnp.int32, (TQ, width), 0)
            cond = cond & (ti <= tr)
        sb = jnp.where(cond, s_w, NEG_MASK)
        m_sel = jnp.max(sb, axis=-1, keepdims=True)
        e_sel = jnp.exp(sb - m_sel)
        z_s = jnp.sum(e_sel, axis=-1, keepdims=True)
        u_s = jax.lax.dot_general(e_sel, vw, (((1,), (0,)), ((), ())),
                                  preferred_element_type=jnp.float32)
        beta = g_beta / jnp.where(z_s == 0.0, 1.0, z_s)
        out_ref[0] = rest + beta * u_s

    qcn = T // TQ
    for lo, hi, w, fc in [(0, 0, TQ, True),
                          (1, qcn // 2 - 1, (qcn // 2) * TQ, False),
                          (qcn // 2, qcn - 1, T, False)]:
        if lo > hi:
            continue
        @pl.when((qc >= lo) & (qc <= hi))
        def _bucket(w=w, fc=fc):
            _selected(w, fc)


def _proj_kernel(x_ref, wo_ref, o_ref, m_s):
    for h in range(H):
        m_s[:, h * HD:(h + 1) * HD] = x_ref[h]
    o_ref[...] = jax.lax.dot_general(m_s[...], wo_ref[...],
                                     (((1,), (1,)), ((), ())),
                                     preferred_element_type=jnp.float32)


def kernel(x, W_q, W_k, W_v, W_o, W_ck, W_cv, W_g, b_g):
    x2d = x.reshape(T, D)
    W_qkv = jnp.concatenate([W_q, W_k, W_v], axis=0).T
    Wg_pad = jnp.zeros((128, HD), jnp.float32).at[:3].set(W_g)
    bg_pad = jnp.full((1, 128), -1.0e30, jnp.float32).at[0, :3].set(b_g)
    E = jnp.where(jnp.arange(T)[None, :] // G == jnp.arange(NB)[:, None],
                  BIG, 0.0).astype(jnp.float32)

    q, k, v, gates = pl.pallas_call(
        _qkv_gates_kernel,
        grid=(T // TQ,),
        in_specs=[
            pl.BlockSpec((TQ, D), lambda i: (i, 0)),
            pl.BlockSpec((D, 3 * D), lambda i: (0, 0)),
            pl.BlockSpec((128, HD), lambda i: (0, 0)),
            pl.BlockSpec((1, 128), lambda i: (0, 0)),
        ],
        out_specs=[
            pl.BlockSpec((H, TQ, HD), lambda i: (0, i, 0)),
            pl.BlockSpec((H, TQ, HD), lambda i: (0, i, 0)),
            pl.BlockSpec((H, TQ, HD), lambda i: (0, i, 0)),
            pl.BlockSpec((TQ, 128), lambda i: (i, 0)),
        ],
        out_shape=[
            jax.ShapeDtypeStruct((H, T, HD), jnp.float32),
            jax.ShapeDtypeStruct((H, T, HD), jnp.float32),
            jax.ShapeDtypeStruct((H, T, HD), jnp.float32),
            jax.ShapeDtypeStruct((T, 128), jnp.float32),
        ],
        interpret=_INTERPRET,
    )(x2d, W_qkv, Wg_pad, bg_pad)

    out3 = pl.pallas_call(
        _attn_kernel,
        grid=(H, T // TQ),
        in_specs=[
            pl.BlockSpec((1, TQ, HD), lambda h, qc: (h, qc, 0)),
            pl.BlockSpec((1, T, HD), lambda h, qc: (h, 0, 0)),
            pl.BlockSpec((1, T, HD), lambda h, qc: (h, 0, 0)),
            pl.BlockSpec((TQ, 128), lambda h, qc: (qc, 0)),
            pl.BlockSpec((HD, G * HD), lambda h, qc: (0, 0)),
            pl.BlockSpec((HD, G * HD), lambda h, qc: (0, 0)),
            pl.BlockSpec((NB, T), lambda h, qc: (0, 0)),
        ],
        out_specs=pl.BlockSpec((1, TQ, HD), lambda h, qc: (h, qc, 0)),
        out_shape=jax.ShapeDtypeStruct((H, T, HD), jnp.float32),
        scratch_shapes=[
            pltpu.VMEM((NB, HD), jnp.float32),
            pltpu.VMEM((NB, HD), jnp.float32),
        ],
        interpret=_INTERPRET,
    )(q, k, v, gates, W_ck, W_cv, E)

    out = pl.pallas_call(
        _proj_kernel,
        grid=(T // TQ,),
        in_specs=[
            pl.BlockSpec((H, TQ, HD), lambda i: (0, i, 0)),
            pl.BlockSpec((D, D), lambda i: (0, 0)),
        ],
        out_specs=pl.BlockSpec((TQ, D), lambda i: (i, 0)),
        out_shape=jax.ShapeDtypeStruct((T, D), jnp.float32),
        scratch_shapes=[pltpu.VMEM((TQ, D), jnp.float32)],
        interpret=_INTERPRET,
    )(out3, W_o)

    return out.reshape(B, T, D)


# speedup vs baseline: 1.0747x; 1.0747x over previous
import math

import jax
import jax.numpy as jnp
from jax.experimental import pallas as pl
from jax.experimental.pallas import tpu as pltpu

B, T, D = 1, 2048, 768
H, HD = 12, 64
G, R, WIN = 16, 4, 64
NB = T // G
SCALE = math.sqrt(HD)
HALF = WIN // 2
TQ = 512
TK = 256
PRE = 128
BAND = TQ + PRE
NEG_TOPK = -1.0e30
NEG_TAKEN = -3.0e38
NEG_MASK = -3.0e38
BIG = 2.0 ** 100

_INTERPRET = False


def _safe_sm(s):
    m = jnp.max(s, axis=-1, keepdims=True)
    m = jnp.where(m > -jnp.inf, m, 0.0)
    e = jnp.exp(s - m)
    z = jnp.sum(e, axis=-1, keepdims=True)
    return e / jnp.where(z == 0.0, 1.0, z)


def _qkv_gates_kernel(x_ref, w_ref, wg_ref, bg_ref,
                      q_ref, k_ref, v_ref, g_ref):
    qkv = jax.lax.dot_general(x_ref[...], w_ref[...], (((1,), (0,)), ((), ())),
                              preferred_element_type=jnp.float32)
    qm = jnp.zeros((TQ, HD), jnp.float32)
    for h in range(H):
        qh = qkv[:, h * HD:(h + 1) * HD]
        q_ref[h] = qh
        qm = qm + qh
        k_ref[h] = qkv[:, D + h * HD:D + (h + 1) * HD]
        v_ref[h] = qkv[:, 2 * D + h * HD:2 * D + (h + 1) * HD]
    qm = qm * (1.0 / H)
    glog = jax.lax.dot_general(qm, wg_ref[...], (((1,), (1,)), ((), ())),
                               preferred_element_type=jnp.float32)
    glog = glog + bg_ref[...]
    m = jnp.max(glog, axis=-1, keepdims=True)
    e = jnp.exp(glog - m)
    g_ref[...] = e / jnp.sum(e, axis=-1, keepdims=True)


def _attn_kernel(q_ref, k_ref, v_ref, g_ref, wck_ref, wcv_ref, e_ref,
                 out_ref, kc_s, vc_s):
    qc = pl.program_id(1)

    @pl.when(qc == 0)
    def _compress():
        k3 = k_ref[0].reshape(NB, G, HD)
        v3 = v_ref[0].reshape(NB, G, HD)
        kc = jnp.zeros((NB, HD), jnp.float32)
        vc = jnp.zeros((NB, HD), jnp.float32)
        for g in range(G):
            wck_g = wck_ref[:, g * HD:(g + 1) * HD]
            wcv_g = wcv_ref[:, g * HD:(g + 1) * HD]
            kc = kc + jax.lax.dot_general(k3[:, g, :], wck_g,
                                          (((1,), (1,)), ((), ())),
                                          preferred_element_type=jnp.float32)
            vc = vc + jax.lax.dot_general(v3[:, g, :], wcv_g,
                                          (((1,), (1,)), ((), ())),
                                          preferred_element_type=jnp.float32)
        kc_s[...] = kc
        vc_s[...] = vc

    q = q_ref[0]
    qs = q * (1.0 / SCALE)
    sc = jax.lax.dot_general(qs, kc_s[...], (((1,), (1,)), ((), ())),
                             preferred_element_type=jnp.float32)
    bi = jax.lax.broadcasted_iota(jnp.int32, (TQ, NB), 1)
    ri = jax.lax.broadcasted_iota(jnp.int32, (TQ, NB), 0) + qc * TQ
    valid = ri >= (bi + 1) * G
    sc_m = jnp.where(valid, sc, -jnp.inf)
    mc = jnp.max(sc_m, axis=-1, keepdims=True)
    mc = jnp.where(mc > -jnp.inf, mc, 0.0)
    e_c = jnp.exp(sc_m - mc)
    z_c = jnp.sum(e_c, axis=-1, keepdims=True)
    u_c = jax.lax.dot_general(e_c, vc_s[...], (((1,), (0,)), ((), ())),
                              preferred_element_type=jnp.float32)
    s = jnp.where(valid, sc,
                  NEG_TOPK * (1.0 + bi.astype(jnp.float32) * (1.0 / 1024.0)))
    bmask = jnp.zeros((TQ, NB), jnp.float32)
    for _ in range(R):
        m = jnp.max(s, axis=-1, keepdims=True)
        hit = s >= m
        bmask = jnp.where(hit, 1.0, bmask)
        s = jnp.where(hit, NEG_TAKEN, s)
    start = jnp.maximum(qc * TQ - PRE, 0)
    k_b = k_ref[0, pl.ds(start, BAND), :]
    v_b = v_ref[0, pl.ds(start, BAND), :]
    s_b = jax.lax.dot_general(qs, k_b, (((1,), (1,)), ((), ())),
                              preferred_element_type=jnp.float32)
    ci = jax.lax.broadcasted_iota(jnp.int32, (TQ, BAND), 1) + start
    rw = jax.lax.broadcasted_iota(jnp.int32, (TQ, BAND), 0) + qc * TQ
    s_b = jnp.where((ci <= rw) & (ci >= rw - HALF), s_b, NEG_MASK)
    m_l = jnp.max(s_b, axis=-1, keepdims=True)
    e_b = jnp.exp(s_b - m_l)
    z_l = jnp.sum(e_b, axis=-1, keepdims=True)
    out_l = jax.lax.dot_general(e_b, v_b, (((1,), (0,)), ((), ())),
                                preferred_element_type=jnp.float32)
    g = g_ref[...]
    alpha = g[:, 0:1] / jnp.where(z_c == 0.0, 1.0, z_c)
    g_beta = g[:, 1:2]
    gamma_out_l = (g[:, 2:3] / z_l) * out_l
    rest = alpha * u_c + gamma_out_l

    def _selected(width, fill_causal):
        kw = k_ref[0, 0:width, :]
        vw = v_ref[0, 0:width, :]
        s_w = jax.lax.dot_general(qs, kw, (((1,), (1,)), ((), ())),
                                  preferred_element_type=jnp.float32)
        selm = jax.lax.dot_general(bmask.astype(jnp.bfloat16),
                                   e_ref[:, 0:width],
                                   (((1,), (0,)), ((), ())),
                                   preferred_element_type=jnp.float32)
        cond = selm > 0.5
        if fill_causal:
            ti = jax.lax.broadcasted_iota(jnp.int32, (TQ, width), 1)
            tr = jax.lax.broadcasted_iota(jnp.int32, (TQ, width), 0)
            cond = cond & (ti <= tr)
        sb = jnp.where(cond, s_w, NEG_MASK)
        m_sel = jnp.max(sb, axis=-1, keepdims=True)
        e_sel = jnp.exp(sb - m_sel)
        z_s = jnp.sum(e_sel, axis=-1, keepdims=True)
        u_s = jax.lax.dot_general(e_sel, vw, (((1,), (0,)), ((), ())),
                                  preferred_element_type=jnp.float32)
        beta = g_beta / jnp.where(z_s == 0.0, 1.0, z_s)
        out_ref[0] = rest + beta * u_s

    qcn = T // TQ
    for lo, hi, w, fc in [(0, 0, TQ, True),
                          (1, qcn // 2 - 1, (qcn // 2) * TQ, False),
                          (qcn // 2, qcn - 1, T, False)]:
        if lo > hi:
            continue
        @pl.when((qc >= lo) & (qc <= hi))
        def _bucket(w=w, fc=fc):
            _selected(w, fc)


def _proj_kernel(x_ref, wo_ref, o_ref, m_s):
    for h in range(H):
        m_s[:, h * HD:(h + 1) * HD] = x_ref[h]
    o_ref[...] = jax.lax.dot_general(m_s[...], wo_ref[...],
                                     (((1,), (1,)), ((), ())),
                                     preferred_element_type=jnp.float32)


def kernel(x, W_q, W_k, W_v, W_o, W_ck, W_cv, W_g, b_g):
    x2d = x.reshape(T, D)
    W_qkv = jnp.concatenate([W_q, W_k, W_v], axis=0).T
    Wg_pad = jnp.zeros((128, HD), jnp.float32).at[:3].set(W_g)
    bg_pad = jnp.full((1, 128), -1.0e30, jnp.float32).at[0, :3].set(b_g)
    E = jnp.where(jnp.arange(T)[None, :] // G == jnp.arange(NB)[:, None],
                  1.0, 0.0).astype(jnp.bfloat16)

    q, k, v, gates = pl.pallas_call(
        _qkv_gates_kernel,
        grid=(T // TQ,),
        in_specs=[
            pl.BlockSpec((TQ, D), lambda i: (i, 0)),
            pl.BlockSpec((D, 3 * D), lambda i: (0, 0)),
            pl.BlockSpec((128, HD), lambda i: (0, 0)),
            pl.BlockSpec((1, 128), lambda i: (0, 0)),
        ],
        out_specs=[
            pl.BlockSpec((H, TQ, HD), lambda i: (0, i, 0)),
            pl.BlockSpec((H, TQ, HD), lambda i: (0, i, 0)),
            pl.BlockSpec((H, TQ, HD), lambda i: (0, i, 0)),
            pl.BlockSpec((TQ, 128), lambda i: (i, 0)),
        ],
        out_shape=[
            jax.ShapeDtypeStruct((H, T, HD), jnp.float32),
            jax.ShapeDtypeStruct((H, T, HD), jnp.float32),
            jax.ShapeDtypeStruct((H, T, HD), jnp.float32),
            jax.ShapeDtypeStruct((T, 128), jnp.float32),
        ],
        interpret=_INTERPRET,
    )(x2d, W_qkv, Wg_pad, bg_pad)

    out3 = pl.pallas_call(
        _attn_kernel,
        grid=(H, T // TQ),
        in_specs=[
            pl.BlockSpec((1, TQ, HD), lambda h, qc: (h, qc, 0)),
            pl.BlockSpec((1, T, HD), lambda h, qc: (h, 0, 0)),
            pl.BlockSpec((1, T, HD), lambda h, qc: (h, 0, 0)),
            pl.BlockSpec((TQ, 128), lambda h, qc: (qc, 0)),
            pl.BlockSpec((HD, G * HD), lambda h, qc: (0, 0)),
            pl.BlockSpec((HD, G * HD), lambda h, qc: (0, 0)),
            pl.BlockSpec((NB, T), lambda h, qc: (0, 0)),
        ],
        out_specs=pl.BlockSpec((1, TQ, HD), lambda h, qc: (h, qc, 0)),
        out_shape=jax.ShapeDtypeStruct((H, T, HD), jnp.float32),
        scratch_shapes=[
            pltpu.VMEM((NB, HD), jnp.float32),
            pltpu.VMEM((NB, HD), jnp.float32),
        ],
        interpret=_INTERPRET,
    )(q, k, v, gates, W_ck, W_cv, E)

    out = pl.pallas_call(
        _proj_kernel,
        grid=(T // TQ,),
        in_specs=[
            pl.BlockSpec((H, TQ, HD), lambda i: (0, i, 0)),
            pl.BlockSpec((D, D), lambda i: (0, 0)),
        ],
        out_specs=pl.BlockSpec((TQ, D), lambda i: (i, 0)),
        out_shape=jax.ShapeDtypeStruct((T, D), jnp.float32),
        scratch_shapes=[pltpu.VMEM((TQ, D), jnp.float32)],
        interpret=_INTERPRET,
    )(out3, W_o)

    return out.reshape(B, T, D)


# speedup vs baseline: 1.0827x; 1.0074x over previous
import math

import jax
import jax.numpy as jnp
from jax.experimental import pallas as pl
from jax.experimental.pallas import tpu as pltpu

B, T, D = 1, 2048, 768
H, HD = 12, 64
G, R, WIN = 16, 4, 64
NB = T // G
SCALE = math.sqrt(HD)
HALF = WIN // 2
TQ = 512
TK = 256
PRE = 128
BAND = TQ + PRE
NEG_TOPK = -1.0e30
NEG_TAKEN = -3.0e38
NEG_MASK = -3.0e38
BIG = 2.0 ** 100

_INTERPRET = False


def _safe_sm(s):
    m = jnp.max(s, axis=-1, keepdims=True)
    m = jnp.where(m > -jnp.inf, m, 0.0)
    e = jnp.exp(s - m)
    z = jnp.sum(e, axis=-1, keepdims=True)
    return e / jnp.where(z == 0.0, 1.0, z)


def _qkv_gates_kernel(x_ref, w_ref, wg_ref, bg_ref,
                      q_ref, k_ref, v_ref, g_ref):
    qkv = jax.lax.dot_general(x_ref[...], w_ref[...], (((1,), (0,)), ((), ())),
                              preferred_element_type=jnp.float32)
    qm = jnp.zeros((TQ, HD), jnp.float32)
    for h in range(H):
        qh = qkv[:, h * HD:(h + 1) * HD]
        q_ref[h] = qh
        qm = qm + qh
        k_ref[h] = qkv[:, D + h * HD:D + (h + 1) * HD]
        v_ref[h] = qkv[:, 2 * D + h * HD:2 * D + (h + 1) * HD]
    qm = qm * (1.0 / H)
    glog = jax.lax.dot_general(qm, wg_ref[...], (((1,), (1,)), ((), ())),
                               preferred_element_type=jnp.float32)
    glog = glog + bg_ref[...]
    m = jnp.max(glog, axis=-1, keepdims=True)
    e = jnp.exp(glog - m)
    g_ref[...] = e / jnp.sum(e, axis=-1, keepdims=True)


def _attn_kernel(q_ref, k_ref, v_ref, g_ref, wck_ref, wcv_ref, e_ref,
                 out_ref, kc_s, vc_s, vext_s):
    qc = pl.program_id(1)

    @pl.when(qc == 0)
    def _vext():
        vext_s[:, 0:HD] = v_ref[0]
        vext_s[:, HD:2 * HD] = jnp.ones((T, HD), jnp.float32)

    @pl.when(qc == 0)
    def _compress():
        k3 = k_ref[0].reshape(NB, G, HD)
        v3 = v_ref[0].reshape(NB, G, HD)
        kc = jnp.zeros((NB, HD), jnp.float32)
        vc = jnp.zeros((NB, HD), jnp.float32)
        for g in range(G):
            wck_g = wck_ref[:, g * HD:(g + 1) * HD]
            wcv_g = wcv_ref[:, g * HD:(g + 1) * HD]
            kc = kc + jax.lax.dot_general(k3[:, g, :], wck_g,
                                          (((1,), (1,)), ((), ())),
                                          preferred_element_type=jnp.float32)
            vc = vc + jax.lax.dot_general(v3[:, g, :], wcv_g,
                                          (((1,), (1,)), ((), ())),
                                          preferred_element_type=jnp.float32)
        kc_s[...] = kc
        vc_s[...] = vc

    q = q_ref[0]
    qs = q * (1.0 / SCALE)
    sc = jax.lax.dot_general(qs, kc_s[...], (((1,), (1,)), ((), ())),
                             preferred_element_type=jnp.float32)
    bi = jax.lax.broadcasted_iota(jnp.int32, (TQ, NB), 1)
    ri = jax.lax.broadcasted_iota(jnp.int32, (TQ, NB), 0) + qc * TQ
    valid = ri >= (bi + 1) * G
    sc_m = jnp.where(valid, sc, -jnp.inf)
    mc = jnp.max(sc_m, axis=-1, keepdims=True)
    mc = jnp.where(mc > -jnp.inf, mc, 0.0)
    e_c = jnp.exp(sc_m - mc)
    z_c = jnp.sum(e_c, axis=-1, keepdims=True)
    u_c = jax.lax.dot_general(e_c, vc_s[...], (((1,), (0,)), ((), ())),
                              preferred_element_type=jnp.float32)
    s = jnp.where(valid, sc,
                  NEG_TOPK * (1.0 + bi.astype(jnp.float32) * (1.0 / 1024.0)))
    bmask = jnp.zeros((TQ, NB), jnp.float32)
    for _ in range(R):
        m = jnp.max(s, axis=-1, keepdims=True)
        hit = s >= m
        bmask = jnp.where(hit, 1.0, bmask)
        s = jnp.where(hit, NEG_TAKEN, s)
    start = jnp.maximum(qc * TQ - PRE, 0)
    k_b = k_ref[0, pl.ds(start, BAND), :]
    v_b = v_ref[0, pl.ds(start, BAND), :]
    s_b = jax.lax.dot_general(qs, k_b, (((1,), (1,)), ((), ())),
                              preferred_element_type=jnp.float32)
    ci = jax.lax.broadcasted_iota(jnp.int32, (TQ, BAND), 1) + start
    rw = jax.lax.broadcasted_iota(jnp.int32, (TQ, BAND), 0) + qc * TQ
    s_b = jnp.where((ci <= rw) & (ci >= rw - HALF), s_b, NEG_MASK)
    m_l = jnp.max(s_b, axis=-1, keepdims=True)
    e_b = jnp.exp(s_b - m_l)
    z_l = jnp.sum(e_b, axis=-1, keepdims=True)
    out_l = jax.lax.dot_general(e_b, v_b, (((1,), (0,)), ((), ())),
                                preferred_element_type=jnp.float32)
    g = g_ref[...]
    alpha = g[:, 0:1] / jnp.where(z_c == 0.0, 1.0, z_c)
    g_beta = g[:, 1:2]
    gamma_out_l = (g[:, 2:3] / z_l) * out_l
    rest = alpha * u_c + gamma_out_l

    def _selected(width, fill_causal):
        kw = k_ref[0, 0:width, :]
        vw = v_ref[0, 0:width, :]
        s_w = jax.lax.dot_general(qs, kw, (((1,), (1,)), ((), ())),
                                  preferred_element_type=jnp.float32)
        selm = jax.lax.dot_general(bmask.astype(jnp.bfloat16),
                                   e_ref[:, 0:width],
                                   (((1,), (0,)), ((), ())),
                                   preferred_element_type=jnp.float32)
        cond = selm > 0.5
        if fill_causal:
            ti = jax.lax.broadcasted_iota(jnp.int32, (TQ, width), 1)
            tr = jax.lax.broadcasted_iota(jnp.int32, (TQ, width), 0)
            cond = cond & (ti <= tr)
        sb = jnp.where(cond, s_w, NEG_MASK)
        m_sel = jnp.max(sb, axis=-1, keepdims=True)
        e_sel = jnp.exp(sb - m_sel)
        u_ext = jax.lax.dot_general(e_sel, vext_s[0:width, :],
                                    (((1,), (0,)), ((), ())),
                                    preferred_element_type=jnp.float32)
        u_s = u_ext[:, 0:HD]
        z_s = u_ext[:, HD:HD + 1]
        beta = g_beta / jnp.where(z_s == 0.0, 1.0, z_s)
        out_ref[0] = rest + beta * u_s

    qcn = T // TQ
    for lo, hi, w, fc in [(0, 0, TQ, True),
                          (1, qcn // 2 - 1, (qcn // 2) * TQ, False),
                          (qcn // 2, qcn - 1, T, False)]:
        if lo > hi:
            continue
        @pl.when((qc >= lo) & (qc <= hi))
        def _bucket(w=w, fc=fc):
            _selected(w, fc)


def _proj_kernel(x_ref, wo_ref, o_ref, m_s):
    for h in range(H):
        m_s[:, h * HD:(h + 1) * HD] = x_ref[h]
    o_ref[...] = jax.lax.dot_general(m_s[...], wo_ref[...],
                                     (((1,), (1,)), ((), ())),
                                     preferred_element_type=jnp.float32)


def kernel(x, W_q, W_k, W_v, W_o, W_ck, W_cv, W_g, b_g):
    x2d = x.reshape(T, D)
    W_qkv = jnp.concatenate([W_q, W_k, W_v], axis=0).T
    Wg_pad = jnp.zeros((128, HD), jnp.float32).at[:3].set(W_g)
    bg_pad = jnp.full((1, 128), -1.0e30, jnp.float32).at[0, :3].set(b_g)
    E = jnp.where(jnp.arange(T)[None, :] // G == jnp.arange(NB)[:, None],
                  1.0, 0.0).astype(jnp.bfloat16)

    q, k, v, gates = pl.pallas_call(
        _qkv_gates_kernel,
        grid=(T // TQ,),
        in_specs=[
            pl.BlockSpec((TQ, D), lambda i: (i, 0)),
            pl.BlockSpec((D, 3 * D), lambda i: (0, 0)),
            pl.BlockSpec((128, HD), lambda i: (0, 0)),
            pl.BlockSpec((1, 128), lambda i: (0, 0)),
        ],
        out_specs=[
            pl.BlockSpec((H, TQ, HD), lambda i: (0, i, 0)),
            pl.BlockSpec((H, TQ, HD), lambda i: (0, i, 0)),
            pl.BlockSpec((H, TQ, HD), lambda i: (0, i, 0)),
            pl.BlockSpec((TQ, 128), lambda i: (i, 0)),
        ],
        out_shape=[
            jax.ShapeDtypeStruct((H, T, HD), jnp.float32),
            jax.ShapeDtypeStruct((H, T, HD), jnp.float32),
            jax.ShapeDtypeStruct((H, T, HD), jnp.float32),
            jax.ShapeDtypeStruct((T, 128), jnp.float32),
        ],
        interpret=_INTERPRET,
    )(x2d, W_qkv, Wg_pad, bg_pad)

    out3 = pl.pallas_call(
        _attn_kernel,
        grid=(H, T // TQ),
        in_specs=[
            pl.BlockSpec((1, TQ, HD), lambda h, qc: (h, qc, 0)),
            pl.BlockSpec((1, T, HD), lambda h, qc: (h, 0, 0)),
            pl.BlockSpec((1, T, HD), lambda h, qc: (h, 0, 0)),
            pl.BlockSpec((TQ, 128), lambda h, qc: (qc, 0)),
            pl.BlockSpec((HD, G * HD), lambda h, qc: (0, 0)),
            pl.BlockSpec((HD, G * HD), lambda h, qc: (0, 0)),
            pl.BlockSpec((NB, T), lambda h, qc: (0, 0)),
        ],
        out_specs=pl.BlockSpec((1, TQ, HD), lambda h, qc: (h, qc, 0)),
        out_shape=jax.ShapeDtypeStruct((H, T, HD), jnp.float32),
        scratch_shapes=[
            pltpu.VMEM((NB, HD), jnp.float32),
            pltpu.VMEM((NB, HD), jnp.float32),
            pltpu.VMEM((T, 2 * HD), jnp.float32),
        ],
        interpret=_INTERPRET,
    )(q, k, v, gates, W_ck, W_cv, E)

    out = pl.pallas_call(
        _proj_kernel,
        grid=(T // TQ,),
        in_specs=[
            pl.BlockSpec((H, TQ, HD), lambda i: (0, i, 0)),
            pl.BlockSpec((D, D), lambda i: (0, 0)),
        ],
        out_specs=pl.BlockSpec((TQ, D), lambda i: (i, 0)),
        out_shape=jax.ShapeDtypeStruct((T, D), jnp.float32),
        scratch_shapes=[pltpu.VMEM((TQ, D), jnp.float32)],
        interpret=_INTERPRET,
    )(out3, W_o)

    return out.reshape(B, T, D)
